# baseline (speedup 1.0000x reference)
import functools

import jax
import jax.numpy as jnp
from jax import lax
from jax.experimental import pallas as pl
from jax.experimental.pallas import tpu as pltpu
from jax.experimental.pallas import tpu_sc as plsc

N_NODES = 10000
D_IN = 128
HIDDEN = 128
LABEL_DIM = 40
LABEL_PAD = 48

NC = 2
NS = 16
K = 128


def _broadcast_lane(v16, lane):
    return lax.gather(
        v16, jnp.full((16, 1), lane, jnp.int32),
        lax.GatherDimensionNumbers(
            offset_dims=(), collapsed_slice_dims=(0,), start_index_map=(0,)),
        (1,), mode=lax.GatherScatterMode.PROMISE_IN_BOUNDS)


def _make_sc_spmm(n_pad, d, e_pad, split):
    d_io = d // 2 if split else d
    n_workers = NS if split else NC * NS
    per_w = e_pad // n_workers
    n_chunks = per_w // K
    assert n_chunks % 8 == 0
    n_blocks = n_chunks // 8
    rows_per_tile = n_pad // NS
    full, rem = divmod(rows_per_tile, K)
    mesh = plsc.VectorSubcoreMesh(core_axis_name="c", subcore_axis_name="s")

    @functools.partial(
        pl.kernel,
        out_type=jax.ShapeDtypeStruct((NC, n_pad, d_io), jnp.float32),
        mesh=mesh,
        scratch_types=[
            pltpu.VMEM((2, 8, 2, K), jnp.int32),
            pltpu.VMEM((2, 8, K), jnp.float32),
            pltpu.VMEM((4, K, d_io), jnp.float32),
            pltpu.VMEM((2, K, d_io), jnp.float32),
            pltpu.VMEM_SHARED((n_pad, d_io), jnp.float32),
            pltpu.SemaphoreType.DMA,
            pltpu.SemaphoreType.DMA,
            pltpu.SemaphoreType.DMA,
            pltpu.SemaphoreType.DMA,
            pltpu.SemaphoreType.DMA,
            pltpu.SemaphoreType.DMA,
            pltpu.SemaphoreType.DMA,
            pltpu.SemaphoreType.DMA,
            pltpu.SemaphoreType.DMA,
            pltpu.SemaphoreType.DMA,
        ],
        compiler_params=pltpu.CompilerParams(use_tc_tiling_on_sc=False),
    )
    def spmm(pk_hbm, w_hbm, h_hbm, out_hbm, idx_v, w_v, grow_v, srow_v, acc_sh,
             isem0, isem1, wsem0, wsem1, g0, g1, g2, g3, s0, s1):
        c = lax.axis_index("c")
        s = lax.axis_index("s")
        wid = s if split else s * NC + c
        bbase = wid * n_blocks
        row0 = s * rows_per_tile
        isem = (isem0, isem1)
        wsem = (wsem0, wsem1)
        gsem = (g0, g1, g2, g3)
        ssem = (s0, s1)

        def iblk_start_dyn(blk, p):
            pltpu.async_copy(
                pk_hbm.at[pl.ds((bbase + blk) * 8, 8)], idx_v.at[p], isem[p])
            pltpu.async_copy(
                w_hbm.at[pl.ds((bbase + blk) * 8, 8)], w_v.at[p], wsem[p])

        def iblk_wait(p):
            pltpu.make_async_copy(
                pk_hbm.at[pl.ds(0, 8)], idx_v.at[p], isem[p]).wait()
            pltpu.make_async_copy(
                w_hbm.at[pl.ds(0, 8)], w_v.at[p], wsem[p]).wait()
            if split:
                for rr in range(8):
                    for gg in range(K // 16):
                        sl = pl.ds(gg * 16, 16)
                        idx_v[p, rr, 0, sl] = idx_v[p, rr, 0, sl] * 2 + c

        def gather_start(p, r8, gb):
            pltpu.async_copy(
                h_hbm.at[idx_v.at[p, r8, 0]], grow_v.at[gb], gsem[gb])

        def gather_wait(p, r8, gb):
            pltpu.make_async_copy(
                h_hbm.at[idx_v.at[p, r8, 0]], grow_v.at[gb], gsem[gb]).wait()

        def scatter_start(p, r8, sb):
            pltpu.async_copy(
                srow_v.at[sb], acc_sh.at[idx_v.at[p, r8, 1]], ssem[sb],
                add=True)

        def scatter_wait(p, r8, sb):
            pltpu.make_async_copy(
                srow_v.at[sb], acc_sh.at[idx_v.at[p, r8, 1]], ssem[sb]).wait()

        def scale(p, r8, gb, sb):
            def edge(e, inner):
                g = e // 16
                lane = e - g * 16
                w16 = w_v[p, r8, pl.ds(g * 16, 16)]
                wb = _broadcast_lane(w16, lane)
                for j in range(d_io // 16):
                    sl = pl.ds(j * 16, 16)
                    srow_v[sb, e, sl] = grow_v[gb, e, sl] * wb
                return inner
            lax.fori_loop(0, K, edge, 0)

        def zrow(i, carry):
            for j in range(d_io // 16):
                srow_v[0, i, pl.ds(j * 16, 16)] = jnp.zeros((16,), jnp.float32)
            return carry
        lax.fori_loop(0, K, zrow, 0)
        for q in range(full):
            pltpu.sync_copy(srow_v.at[0], acc_sh.at[pl.ds(row0 + q * K, K)])
        if rem:
            pltpu.sync_copy(srow_v.at[0, pl.ds(0, rem)],
                            acc_sh.at[pl.ds(row0 + full * K, rem)])
        plsc.subcore_barrier()

        iblk_start_dyn(0, 0)
        iblk_wait(0)
        for r in range(4):
            gather_start(0, r, r)

        n_pairs = n_blocks // 2

        def do_block(q, bb):
            p = bb
            np_ = 1 - bb

            def blk_lt(x):
                return q * 2 + bb < x

            for r in range(8):
                gb = r % 4
                sb = r % 2

                if r >= 2:
                    scatter_wait(p, r - 2, sb)
                elif bb == 1:
                    scatter_wait(np_, r + 6, sb)
                else:

                    @pl.when(q >= 1)
                    def _():
                        scatter_wait(np_, r + 6, sb)

                if r == 2:
                    if bb == 0:
                        iblk_start_dyn(q * 2 + 1, np_)
                    else:

                        @pl.when(blk_lt(n_blocks - 1))
                        def _():
                            iblk_start_dyn(q * 2 + 2, np_)

                gather_wait(p, r, gb)
                scale(p, r, gb, sb)
                scatter_start(p, r, sb)

                if r < 4:
                    gather_start(p, r + 4, gb)
                elif bb == 0:
                    if r == 4:
                        iblk_wait(np_)
                    gather_start(np_, r - 4, gb)
                else:

                    @pl.when(blk_lt(n_blocks - 1))
                    def _():
                        if r == 4:
                            iblk_wait(np_)
                        gather_start(np_, r - 4, gb)

        def pair_body(q, carry):
            do_block(q, 0)
            do_block(q, 1)
            return carry
        lax.fori_loop(0, n_pairs, pair_body, 0)
        lp = (n_blocks - 1) % 2
        scatter_wait(lp, 6, 0)
        scatter_wait(lp, 7, 1)
        plsc.subcore_barrier()

        pltpu.sync_copy(acc_sh.at[pl.ds(row0, rows_per_tile)],
                        out_hbm.at[c, pl.ds(row0, rows_per_tile)])

    return spmm


def _mm_body(x_ref, w_ref, o_ref):
    o_ref[...] = jnp.dot(x_ref[...], w_ref[...],
                         preferred_element_type=jnp.float32)


def _fuse1_body(p_ref, b_ref, w_ref, o_ref):
    hcat = jnp.concatenate([p_ref[0], p_ref[1]], axis=1)
    h = jnp.maximum(hcat + b_ref[...], 0.0)
    o_ref[...] = jnp.dot(h, w_ref[...], preferred_element_type=jnp.float32)


def _fuse2_body(p_ref, b_ref, o_ref):
    s = p_ref[0] + p_ref[1] + b_ref[...]
    logits = s[:, :LABEL_DIM]
    m = jnp.max(logits, axis=1, keepdims=True)
    z = logits - m
    lse = jnp.log(jnp.sum(jnp.exp(z), axis=1, keepdims=True))
    o_ref[...] = z - lse


def kernel(x, edge_index, edge_weight, W1, b1, W2, b2):
    n = x.shape[0]
    e = edge_weight.shape[0]
    chunk_span = NC * NS * K * 2
    e_pad = ((e + chunk_span - 1) // chunk_span) * chunk_span
    row_span = NS * 8
    n_pad = ((n + row_span - 1) // row_span) * row_span

    src = edge_index[0].astype(jnp.int32)
    dst = edge_index[1].astype(jnp.int32)
    pad = e_pad - e
    if pad:
        src = jnp.pad(src, (0, pad))
        dst = jnp.pad(dst, (0, pad))
        edge_weight = jnp.pad(edge_weight, (0, pad))
    packed = jnp.stack([src.reshape(-1, K), dst.reshape(-1, K)], axis=1)
    wchunk = edge_weight.reshape(-1, K)

    w2p = jnp.pad(W2, ((0, 0), (0, LABEL_PAD - LABEL_DIM)))
    b1r = b1.reshape(1, HIDDEN)
    b2r = jnp.pad(b2, (0, LABEL_PAD - LABEL_DIM)).reshape(1, LABEL_PAD)

    h1 = pl.pallas_call(
        _mm_body,
        out_shape=jax.ShapeDtypeStruct((n, HIDDEN), jnp.float32),
    )(x, W1)

    spmm1 = _make_sc_spmm(n_pad, HIDDEN, e_pad, split=True)
    p1 = spmm1(packed, wchunk, h1.reshape(2 * n, HIDDEN // 2))

    h2 = pl.pallas_call(
        _fuse1_body,
        out_shape=jax.ShapeDtypeStruct((n_pad, LABEL_PAD), jnp.float32),
    )(p1, b1r, w2p)

    spmm2 = _make_sc_spmm(n_pad, LABEL_PAD, e_pad, split=False)
    p2 = spmm2(packed, wchunk, h2)

    out = pl.pallas_call(
        _fuse2_body,
        out_shape=jax.ShapeDtypeStruct((n_pad, LABEL_DIM), jnp.float32),
    )(p2, b2r)
    return out[:n]
